# baseline (speedup 1.0000x reference)
import functools

import jax
import jax.numpy as jnp
from jax import lax
from jax.experimental import pallas as pl
from jax.experimental.pallas import tpu as pltpu
from jax.experimental.pallas import tpu_sc as plsc

NUM_EMBEDDINGS = 1000000
EMBED_DIM = 64
BATCH = 4096
SEQ = 200
EPS = 1e-5

TOTAL_ROWS = BATCH * SEQ
LANES = 16
SUBGATHER = 128


def _lane_perm(v, idx):
    dnums = lax.GatherDimensionNumbers(
        offset_dims=(), collapsed_slice_dims=(0,), start_index_map=(0,))
    return lax.gather(v, idx[:, None], dnums, (1,),
                      mode=lax.GatherScatterMode.PROMISE_IN_BOUNDS)


def _lane_sum(v, xor_idx):
    for idx in xor_idx:
        v = v + _lane_perm(v, idx)
    return v


def _fast_rsqrt(w):
    i = lax.bitcast_convert_type(w, jnp.int32)
    i = jnp.int32(0x5F3759DF) - lax.shift_right_logical(i, 1)
    y = lax.bitcast_convert_type(i, jnp.float32)
    half = jnp.float32(0.5) * w
    for _ in range(3):
        y = y * (jnp.float32(1.5) - half * y * y)
    return y


def _make_sc_kernel(rows_per_w, chunk):
    n_chunks = rows_per_w // chunk
    mesh = plsc.VectorSubcoreMesh(core_axis_name="c", subcore_axis_name="s")

    @functools.partial(
        pl.kernel,
        mesh=mesh,
        out_type=jax.ShapeDtypeStruct((TOTAL_ROWS, EMBED_DIM), jnp.float32),
        scratch_types=[
            pltpu.VMEM((chunk,), jnp.int32),
            pltpu.VMEM((chunk, EMBED_DIM), jnp.float32),
            pltpu.VMEM((EMBED_DIM,), jnp.float32),
            pltpu.VMEM((EMBED_DIM,), jnp.float32),
            pltpu.SemaphoreType.DMA,
        ],
        compiler_params=pltpu.CompilerParams(use_tc_tiling_on_sc=False),
    )
    def sc_kernel(table_hbm, xf_hbm, gamma_hbm, beta_hbm, out_hbm,
                  idx_v, rows_v, gam_v, bet_v, sem):
        wid = lax.axis_index("s") * 2 + lax.axis_index("c")
        base = wid * rows_per_w

        pltpu.sync_copy(gamma_hbm, gam_v)
        pltpu.sync_copy(beta_hbm, bet_v)
        gvecs = [gam_v[pl.ds(q * LANES, LANES)] for q in range(4)]
        bvecs = [bet_v[pl.ds(q * LANES, LANES)] for q in range(4)]

        lane_iota = lax.iota(jnp.int32, LANES)
        xor_idx = [lane_iota ^ k for k in (1, 2, 4, 8)]
        inv_d = jnp.float32(1.0 / EMBED_DIM)

        def chunk_body(c, carry):
            cb = pl.multiple_of(base + c * chunk, 8)
            pltpu.sync_copy(xf_hbm.at[pl.ds(cb, chunk)], idx_v)
            copies = []
            for j in range(chunk // SUBGATHER):
                copies.append(pltpu.async_copy(
                    table_hbm.at[idx_v.at[pl.ds(j * SUBGATHER, SUBGATHER)]],
                    rows_v.at[pl.ds(j * SUBGATHER, SUBGATHER)],
                    sem))
            for cp in copies:
                cp.wait()

            def row_body(r, rcarry):
                v = [rows_v[r, pl.ds(q * LANES, LANES)] for q in range(4)]
                s = (v[0] + v[1]) + (v[2] + v[3])
                ss = (v[0] * v[0] + v[1] * v[1]) + (v[2] * v[2] + v[3] * v[3])
                s = _lane_sum(s, xor_idx)
                ss = _lane_sum(ss, xor_idx)
                mean = s * inv_d
                var = ss * inv_d - mean * mean
                rstd = _fast_rsqrt(var + jnp.float32(EPS))
                for q in range(4):
                    a = rstd * gvecs[q]
                    b = bvecs[q] - mean * a
                    rows_v[r, pl.ds(q * LANES, LANES)] = v[q] * a + b
                return rcarry

            lax.fori_loop(0, chunk, row_body, 0, unroll=2)
            pltpu.sync_copy(rows_v, out_hbm.at[pl.ds(cb, chunk)])
            return carry

        lax.fori_loop(0, n_chunks, chunk_body, 0)

    return sc_kernel


_sc_kernel = _make_sc_kernel(rows_per_w=TOTAL_ROWS // 32, chunk=512)


@jax.jit
def kernel(x, table, gamma, beta):
    xf = x.reshape(-1)
    out = _sc_kernel(table, xf, gamma, beta)
    return out.reshape(BATCH, SEQ, EMBED_DIM)
